# Initial kernel scaffold; baseline (speedup 1.0000x reference)
import jax
import jax.numpy as jnp
from jax.experimental import pallas as pl


def kernel(idx, weight):
    raise NotImplementedError("write your pallas kernel here")


# speedup vs baseline: 14.1055x; 14.1055x over previous
import jax
import jax.numpy as jnp
from jax.experimental import pallas as pl

B, T, V, C = 16384, 200, 10, 3
BT = 1024


def _body(idx_ref, w_ref, out_ref):
    w = w_ref[...]
    g = jnp.dot(w, w.T, preferred_element_type=jnp.float32)

    cmod = jax.lax.broadcasted_iota(jnp.int32, (V, T * V), 1) % V
    jrow = jax.lax.broadcasted_iota(jnp.int32, (V, T * V), 0)
    s_sel = (cmod == jrow).astype(jnp.float32)
    gtile = jnp.dot(g, s_sel, preferred_element_type=jnp.float32)

    rrow = jax.lax.broadcasted_iota(jnp.int32, (T, T * V), 0)
    rcol = jax.lax.broadcasted_iota(jnp.int32, (T, T * V), 1)
    r_exp = (rcol // V == rrow).astype(jnp.float32)
    idxf = idx_ref[...].astype(jnp.float32)
    idx_exp = jnp.dot(idxf, r_exp, preferred_element_type=jnp.float32)

    acc = jnp.broadcast_to(gtile[V - 1 : V, :], idx_exp.shape)
    for k in range(V - 2, -1, -1):
        acc = jnp.where(idx_exp == k, gtile[k : k + 1, :], acc)
    out_ref[...] = acc


def kernel(idx, weight):
    out2d = pl.pallas_call(
        _body,
        grid=(B // BT,),
        in_specs=[
            pl.BlockSpec((BT, T), lambda i: (i, 0)),
            pl.BlockSpec((V, C), lambda i: (0, 0)),
        ],
        out_specs=pl.BlockSpec((BT, T * V), lambda i: (i, 0)),
        out_shape=jax.ShapeDtypeStruct((B, T * V), jnp.float32),
    )(idx, weight)
    return out2d.reshape(B, T, V)
